# speedup vs baseline: 1.8819x; 1.8819x over previous
import math

import jax
import jax.numpy as jnp
from jax.experimental import pallas as pl
from jax.experimental.pallas import tpu as pltpu


def _round_up(a, b):
    return (a + b - 1) // b * b


def _cdiv(a, b):
    return -(-a // b)


def _neg_min(dtype):
    dtype = jnp.dtype(dtype)
    if jnp.issubdtype(dtype, jnp.floating):
        return float("-inf")
    if jnp.issubdtype(dtype, jnp.integer):
        return int(jnp.iinfo(dtype).min)
    raise ValueError(f"unsupported dtype for max pooling: {dtype}")


def _make_body(num_groups, last_valid, min_val):
    def body(x_ref, o_ref):
        m = None
        for g in range(num_groups):
            blk = x_ref[:, g * 128:(g + 1) * 128]
            if g == num_groups - 1 and last_valid < 128:
                lane = jax.lax.broadcasted_iota(jnp.int32, blk.shape, 1)
                blk = jnp.where(lane < last_valid, blk,
                                jnp.full_like(blk, min_val))
            m = blk if m is None else jnp.maximum(m, blk)
        o_ref[...] = jnp.max(m, axis=-1, keepdims=True).astype(o_ref.dtype)

    return body


def _global_max_last_axis(x):
    *lead, L = x.shape
    R = math.prod(lead) if lead else 1
    out_shape = tuple(lead)

    itemsize = jnp.dtype(x.dtype).itemsize
    sub = {4: 8, 2: 16, 1: 32}.get(itemsize, 8)
    Lp = _round_up(L, 128)
    num_groups = Lp // 128
    last_valid = L - (num_groups - 1) * 128

    budget = 8 * 1024 * 1024
    TR = max(sub, min(_round_up(R, sub), 2048,
                      (budget // (Lp * itemsize)) // sub * sub))
    if _cdiv(R, TR) < 2 and R > sub:
        TR = _round_up(_cdiv(R, 2), sub)
    num_r = _cdiv(R, TR)

    xf = x.reshape(R, L)
    out = pl.pallas_call(
        _make_body(num_groups, last_valid, _neg_min(x.dtype)),
        out_shape=jax.ShapeDtypeStruct((R, 1), x.dtype),
        grid=(num_r,),
        in_specs=[pl.BlockSpec((TR, Lp), lambda i: (i, 0))],
        out_specs=pl.BlockSpec((TR, 1), lambda i: (i, 0)),
        compiler_params=pltpu.CompilerParams(
            dimension_semantics=("parallel",),
            vmem_limit_bytes=48 * 1024 * 1024,
        ),
    )(xf)

    return out[:, 0].reshape(out_shape)


def kernel(x):
    return _global_max_last_axis(x)
